# Initial kernel scaffold; baseline (speedup 1.0000x reference)
import jax
import jax.numpy as jnp
from jax.experimental import pallas as pl


def kernel(x, tok_table, pos_table):
    raise NotImplementedError("write your pallas kernel here")


# speedup vs baseline: 3.2828x; 3.2828x over previous
import functools

import jax
import jax.numpy as jnp
from jax import lax
from jax.experimental import pallas as pl
from jax.experimental.pallas import tpu as pltpu
from jax.experimental.pallas import tpu_sc as plsc

VOCAB = 100000
EMBED = 64
B, L = 4096, 200

NC, NS = 2, 16
NW = NC * NS
GCH = 100
TOTAL = B * L
NGATH = TOTAL // GCH
CPW = NGATH // NW // 2

_mesh = plsc.VectorSubcoreMesh(
    core_axis_name="c", subcore_axis_name="s", num_cores=NC, num_subcores=NS
)


@functools.partial(
    pl.kernel,
    out_type=jax.ShapeDtypeStruct((TOTAL, EMBED), jnp.float32),
    mesh=_mesh,
    compiler_params=pltpu.CompilerParams(use_tc_tiling_on_sc=False),
    scratch_types=[
        pltpu.VMEM((2 * CPW, GCH), jnp.int32),
        pltpu.VMEM((L, EMBED), jnp.float32),
        pltpu.VMEM((L, EMBED), jnp.float32),
        pltpu.SemaphoreType.DMA,
    ],
)
def _sc_embed(x_hbm, tok_hbm, pos_hbm, out_hbm, idx_v, pos_v, rows_v, sem):
    wid = lax.axis_index("s") * NC + lax.axis_index("c")
    pltpu.sync_copy(pos_hbm.at[pl.ds(0, L)], pos_v)
    pltpu.sync_copy(x_hbm.at[pl.ds(wid * 2 * CPW, 2 * CPW)], idx_v)

    @pl.loop(0, CPW)
    def _chunk(c):
        g0 = pltpu.async_copy(
            tok_hbm.at[idx_v.at[2 * c]], rows_v.at[pl.ds(0, GCH)], sem
        )
        g1 = pltpu.async_copy(
            tok_hbm.at[idx_v.at[2 * c + 1]], rows_v.at[pl.ds(GCH, GCH)], sem
        )
        g0.wait()
        g1.wait()

        @pl.loop(0, L)
        def _row(r):
            for k in range(EMBED // 16):
                sl = pl.ds(k * 16, 16)
                plsc.addupdate(rows_v.at[r, sl], pos_v[r, sl])

        pltpu.sync_copy(rows_v, out_hbm.at[pl.ds((wid * CPW + c) * L, L)])


def kernel(x, tok_table, pos_table):
    xf = x.reshape(-1).astype(jnp.int32).reshape(NGATH, GCH)
    out = _sc_embed(xf, tok_table, pos_table)
    return out.reshape(B, L, EMBED)


# speedup vs baseline: 3.9731x; 1.2103x over previous
import functools

import jax
import jax.numpy as jnp
from jax import lax
from jax.experimental import pallas as pl
from jax.experimental.pallas import tpu as pltpu
from jax.experimental.pallas import tpu_sc as plsc

VOCAB = 100000
EMBED = 64
B, L = 4096, 200

NC, NS = 2, 16
NW = NC * NS
GCH = 100
TOTAL = B * L
NGATH = TOTAL // GCH
CPW = NGATH // NW // 2

_mesh = plsc.VectorSubcoreMesh(
    core_axis_name="c", subcore_axis_name="s", num_cores=NC, num_subcores=NS
)


@functools.partial(
    pl.kernel,
    out_type=jax.ShapeDtypeStruct((TOTAL, EMBED), jnp.float32),
    mesh=_mesh,
    compiler_params=pltpu.CompilerParams(use_tc_tiling_on_sc=False),
    scratch_types=[
        pltpu.VMEM((2 * CPW, GCH), jnp.int32),
        pltpu.VMEM((L, EMBED), jnp.float32),
        pltpu.VMEM((L, EMBED), jnp.float32),
        pltpu.VMEM((L, EMBED), jnp.float32),
        pltpu.SemaphoreType.DMA,
        pltpu.SemaphoreType.DMA,
    ],
)
def _sc_embed(x_hbm, tok_hbm, pos_hbm, out_hbm, idx_v, pos_v, rows0, rows1,
              sem_g, sem_o):
    wid = lax.axis_index("s") * NC + lax.axis_index("c")
    pltpu.sync_copy(pos_hbm.at[pl.ds(0, L)], pos_v)
    pltpu.sync_copy(x_hbm.at[pl.ds(wid * 2 * CPW, 2 * CPW)], idx_v)
    bufs = (rows0, rows1)

    def start_gather(c, buf):
        pltpu.async_copy(tok_hbm.at[idx_v.at[2 * c]], buf.at[pl.ds(0, GCH)], sem_g)
        pltpu.async_copy(
            tok_hbm.at[idx_v.at[2 * c + 1]], buf.at[pl.ds(GCH, GCH)], sem_g
        )

    def wait_gather(buf):
        pltpu.make_async_copy(tok_hbm.at[pl.ds(0, L)], buf, sem_g).wait()

    def wait_out(buf):
        pltpu.make_async_copy(buf, out_hbm.at[pl.ds(0, L)], sem_o).wait()

    start_gather(0, rows0)

    @pl.loop(0, CPW, step=2)
    def _chunk(c0):
        for b in range(2):
            c = c0 + b
            cur, nxt = bufs[b], bufs[1 - b]

            @pl.when(c >= 1)
            def _free_nxt():
                wait_out(nxt)

            @pl.when(c + 1 < CPW)
            def _prefetch():
                start_gather(c + 1, nxt)

            wait_gather(cur)

            @pl.loop(0, L, unroll=4)
            def _row(r):
                for k in range(EMBED // 16):
                    sl = pl.ds(k * 16, 16)
                    plsc.addupdate(cur.at[r, sl], pos_v[r, sl])

            pltpu.async_copy(cur, out_hbm.at[pl.ds((wid * CPW + c) * L, L)], sem_o)

    wait_out(rows1)


def kernel(x, tok_table, pos_table):
    xf = x.reshape(-1).astype(jnp.int32).reshape(NGATH, GCH)
    out = _sc_embed(xf, tok_table, pos_table)
    return out.reshape(B, L, EMBED)
